# baseline (speedup 1.0000x reference)
import jax
import jax.numpy as jnp
from jax import lax
from jax.experimental import pallas as pl
from jax.experimental.pallas import tpu as pltpu
from jax.experimental.pallas import tpu_sc as plsc

N = 10000
E = 320000
D = 128
R = 8

NC = 2
NS = 16
NW = NC * NS
L = 16

C = 16
EPT = 10016
EPAD = EPT * NW
NCHUNK = EPT // C
NA = N + 16
ND = 80


_BM = 400


def _mm_body(x_ref, wcat_ref, wskip_ref, b_ref, y_ref, s_ref):
    x = x_ref[...]
    y_ref[...] = jnp.dot(x, wcat_ref[...], preferred_element_type=jnp.float32)
    s_ref[...] = (
        jnp.dot(x, wskip_ref[...], preferred_element_type=jnp.float32)
        + b_ref[...]
    )


def _premultiply(x, wcat, wskip, b2d):
    return pl.pallas_call(
        _mm_body,
        grid=(N // _BM,),
        in_specs=[
            pl.BlockSpec((_BM, D), lambda i: (i, 0)),
            pl.BlockSpec((D, R * D), lambda i: (0, 0)),
            pl.BlockSpec((D, D), lambda i: (0, 0)),
            pl.BlockSpec((1, D), lambda i: (0, 0)),
        ],
        out_specs=[
            pl.BlockSpec((_BM, R * D), lambda i: (i, 0)),
            pl.BlockSpec((_BM, D), lambda i: (i, 0)),
        ],
        out_shape=[
            jax.ShapeDtypeStruct((N, R * D), jnp.float32),
            jax.ShapeDtypeStruct((N, D), jnp.float32),
        ],
    )(x, wcat, wskip, b2d)


def _sc_body(y_hbm, src_hbm, dst_hbm, wt_hbm, z2d_hbm, oh_hbm,
             acc_out, degw_out,
             src0, src1, dst0, dst1, drow_v, dmod_v, w0, w1, rows0, rows1,
             m_v, m2_v, acc_sh, deg_sh, sem0, sem1, sem2):
    c = lax.axis_index("c")
    s = lax.axis_index("s")
    wid = c * NS + s

    srcs, dsts, ws_, rows_, sems = (
        (src0, src1), (dst0, dst1), (w0, w1), (rows0, rows1), (sem0, sem1)
    )

    def issue(t, b):
        base = wid * EPT + t * C
        pltpu.sync_copy(src_hbm.at[pl.ds(base, C)], srcs[b])
        pltpu.sync_copy(dst_hbm.at[pl.ds(base, C)], dsts[b])
        pltpu.sync_copy(wt_hbm.at[pl.ds(base * R, C * R)], ws_[b])
        pltpu.async_copy(y_hbm.at[srcs[b]], rows_[b], sems[b])

    issue(0, 0)
    issue(1, 1)

    @pl.when(s < 10)
    def _():
        pltpu.sync_copy(z2d_hbm, acc_sh.at[pl.ds(s * 1000, 1000)])

    @pl.when(s == 10)
    def _():
        pltpu.sync_copy(z2d_hbm.at[pl.ds(0, 16)], acc_sh.at[pl.ds(N, 16)])

    @pl.when(s == 11)
    def _():
        pltpu.sync_copy(z2d_hbm.at[pl.ds(0, ND)], deg_sh)

    plsc.subcore_barrier()

    def chunk_pair(tt, _):
        for b in range(2):
            t = tt * 2 + b
            src_v, dst_v, w_v, rows_v, sem = (
                srcs[b], dsts[b], ws_[b], rows_[b], sems[b]
            )
            pltpu.make_async_copy(y_hbm.at[src_v], rows_v, sem).wait()

            dst16 = dst_v[...]
            drow_v[...] = lax.shift_right_logical(dst16, 7)
            dmod_v[...] = jnp.bitwise_and(dst16, 127)
            pltpu.async_copy(oh_hbm.at[dmod_v], m2_v, sem2).wait()

            def pair_body(p, _):
                w16 = w_v[pl.ds(p * 2 * R, L)]
                for e in range(2):
                    i = p * 2 + e
                    m = [None] * (D // L)
                    for r in range(R):
                        wb = jnp.broadcast_to(w16[e * R + r], (L,))
                        for j in range(D // L):
                            v = rows_v[i, pl.ds(r * D + j * L, L)] * wb
                            m[j] = v if r == 0 else m[j] + v
                    for j in range(D // L):
                        m_v[i, pl.ds(j * L, L)] = m[j]
                return 0

            lax.fori_loop(0, C // 2, pair_body, 0, unroll=False)

            pltpu.sync_copy(m_v, acc_sh.at[dst_v], add=True)
            pltpu.sync_copy(m2_v, deg_sh.at[drow_v], add=True)

            @pl.when(t + 2 < NCHUNK)
            def _():
                issue(t + 2, b)
        return 0

    lax.fori_loop(0, NCHUNK // 2, chunk_pair, 0, unroll=False)

    plsc.subcore_barrier()

    @pl.when(s < 10)
    def _():
        pltpu.sync_copy(acc_sh.at[pl.ds(s * 1000, 1000)],
                        acc_out.at[c, pl.ds(s * 1000, 1000)])

    @pl.when(s == 11)
    def _():
        pltpu.sync_copy(deg_sh, degw_out.at[c])


_sc_edge_pass = pl.kernel(
    _sc_body,
    out_type=[
        jax.ShapeDtypeStruct((NC, N, D), jnp.float32),
        jax.ShapeDtypeStruct((NC, ND, D), jnp.float32),
    ],
    mesh=plsc.VectorSubcoreMesh(
        core_axis_name="c", subcore_axis_name="s", num_cores=NC,
        num_subcores=NS,
    ),
    scratch_types=[
        pltpu.VMEM((C,), jnp.int32),
        pltpu.VMEM((C,), jnp.int32),
        pltpu.VMEM((C,), jnp.int32),
        pltpu.VMEM((C,), jnp.int32),
        pltpu.VMEM((C,), jnp.int32),
        pltpu.VMEM((C,), jnp.int32),
        pltpu.VMEM((C * R,), jnp.float32),
        pltpu.VMEM((C * R,), jnp.float32),
        pltpu.VMEM((C, R * D), jnp.float32),
        pltpu.VMEM((C, R * D), jnp.float32),
        pltpu.VMEM((C, D), jnp.float32),
        pltpu.VMEM((C, D), jnp.float32),
        pltpu.VMEM_SHARED((NA, D), jnp.float32),
        pltpu.VMEM_SHARED((ND, D), jnp.float32),
        pltpu.SemaphoreType.DMA,
        pltpu.SemaphoreType.DMA,
        pltpu.SemaphoreType.DMA,
    ],
)


def _combine_body(a0_ref, a1_ref, d_ref, s_ref, o_ref):
    deg = jnp.maximum(d_ref[...], 1.0)
    o_ref[...] = (a0_ref[...] + a1_ref[...]) / deg + s_ref[...]


def _combine(a0, a1, d, skip):
    return pl.pallas_call(
        _combine_body,
        grid=(N // _BM,),
        in_specs=[
            pl.BlockSpec((_BM, D), lambda i: (i, 0)),
            pl.BlockSpec((_BM, D), lambda i: (i, 0)),
            pl.BlockSpec((_BM, 1), lambda i: (i, 0)),
            pl.BlockSpec((_BM, D), lambda i: (i, 0)),
        ],
        out_specs=pl.BlockSpec((_BM, D), lambda i: (i, 0)),
        out_shape=jax.ShapeDtypeStruct((N, D), jnp.float32),
    )(a0, a1, d, skip)


@jax.jit
def kernel(node_feats, edge_weights, rel_fcs, skip_w, skip_b, edge_index):
    wcat = rel_fcs.transpose(1, 0, 2).reshape(D, R * D)
    y, skip = _premultiply(node_feats, wcat, skip_w, skip_b.reshape(1, D))

    npad = EPAD - E
    src = jnp.concatenate([edge_index[0], jnp.zeros((npad,), jnp.int32)])
    dst = jnp.concatenate([edge_index[1], jnp.full((npad,), N, jnp.int32)])
    wt = jnp.concatenate(
        [edge_weights.T, jnp.zeros((npad, R), jnp.float32)]
    ).reshape(EPAD * R)

    z2d = jnp.zeros((1000, D), jnp.float32)
    oh = jnp.eye(D, dtype=jnp.float32)
    acc, degw = _sc_edge_pass(y, src, dst, wt, z2d, oh)

    deg = (degw[0] + degw[1]).reshape(ND * D)[:N].reshape(N, 1)
    return _combine(acc[0], acc[1], deg, skip)


# speedup vs baseline: 1.2747x; 1.2747x over previous
import jax
import jax.numpy as jnp
from jax import lax
from jax.experimental import pallas as pl
from jax.experimental.pallas import tpu as pltpu
from jax.experimental.pallas import tpu_sc as plsc

N = 10000
E = 320000
D = 128
R = 8

NC = 2
NS = 16
NW = NC * NS
L = 16

C = 16
EPT = 10016
EPAD = EPT * NW
NCHUNK = EPT // C
NA = N + 16
ND = 80


_BM = 400


def _mm_body(x_ref, wcat_ref, wskip_ref, b_ref, y_ref, s_ref):
    x = x_ref[...]
    y_ref[...] = jnp.dot(x, wcat_ref[...], preferred_element_type=jnp.float32)
    s_ref[...] = (
        jnp.dot(x, wskip_ref[...], preferred_element_type=jnp.float32)
        + b_ref[...]
    )


def _premultiply(x, wcat, wskip, b2d):
    return pl.pallas_call(
        _mm_body,
        grid=(N // _BM,),
        in_specs=[
            pl.BlockSpec((_BM, D), lambda i: (i, 0)),
            pl.BlockSpec((D, R * D), lambda i: (0, 0)),
            pl.BlockSpec((D, D), lambda i: (0, 0)),
            pl.BlockSpec((1, D), lambda i: (0, 0)),
        ],
        out_specs=[
            pl.BlockSpec((_BM, R * D), lambda i: (i, 0)),
            pl.BlockSpec((_BM, D), lambda i: (i, 0)),
        ],
        out_shape=[
            jax.ShapeDtypeStruct((N, R * D), jnp.float32),
            jax.ShapeDtypeStruct((N, D), jnp.float32),
        ],
    )(x, wcat, wskip, b2d)


def _sc_body(y_hbm, src_hbm, dst_hbm, wt_hbm, z2d_hbm, oh_hbm,
             acc_out, degw_out,
             src0, src1, dst0, dst1, drow_v, dmod0, dmod1, w0, w1,
             rows0, rows1, m_v, oh0, oh1, acc_sh, deg_sh,
             sem0, sem1, sem2, sem3):
    c = lax.axis_index("c")
    s = lax.axis_index("s")
    wid = c * NS + s

    srcs, dsts, ws_, rows_, sems = (
        (src0, src1), (dst0, dst1), (w0, w1), (rows0, rows1), (sem0, sem1)
    )
    ohs, osems, dmods = (oh0, oh1), (sem2, sem3), (dmod0, dmod1)

    def issue(t, b):
        base = wid * EPT + t * C
        pltpu.sync_copy(src_hbm.at[pl.ds(base, C)], srcs[b])
        pltpu.sync_copy(dst_hbm.at[pl.ds(base, C)], dsts[b])
        pltpu.sync_copy(wt_hbm.at[pl.ds(base * R, C * R)], ws_[b])
        pltpu.async_copy(y_hbm.at[srcs[b]], rows_[b], sems[b])
        dmods[b][...] = jnp.bitwise_and(dsts[b][...], 127)
        pltpu.async_copy(oh_hbm.at[dmods[b]], ohs[b], osems[b])

    issue(0, 0)
    issue(1, 1)

    @pl.when(s < 10)
    def _():
        pltpu.sync_copy(z2d_hbm, acc_sh.at[pl.ds(s * 1000, 1000)])

    @pl.when(s == 10)
    def _():
        pltpu.sync_copy(z2d_hbm.at[pl.ds(0, 16)], acc_sh.at[pl.ds(N, 16)])

    @pl.when(s == 11)
    def _():
        pltpu.sync_copy(z2d_hbm.at[pl.ds(0, ND)], deg_sh)

    plsc.subcore_barrier()

    def chunk_pair(tt, _):
        for b in range(2):
            t = tt * 2 + b
            src_v, dst_v, w_v, rows_v, sem = (
                srcs[b], dsts[b], ws_[b], rows_[b], sems[b]
            )
            pltpu.make_async_copy(y_hbm.at[src_v], rows_v, sem).wait()
            pltpu.make_async_copy(oh_hbm.at[dmods[b]], ohs[b], osems[b]).wait()

            drow_v[...] = lax.shift_right_logical(dst_v[...], 7)

            def pair_body(p, _):
                w16 = w_v[pl.ds(p * 2 * R, L)]
                for e in range(2):
                    i = p * 2 + e
                    m = [None] * (D // L)
                    for r in range(R):
                        wb = jnp.broadcast_to(w16[e * R + r], (L,))
                        for j in range(D // L):
                            v = rows_v[i, pl.ds(r * D + j * L, L)] * wb
                            m[j] = v if r == 0 else m[j] + v
                    for j in range(D // L):
                        m_v[i, pl.ds(j * L, L)] = m[j]
                return 0

            lax.fori_loop(0, C // 2, pair_body, 0, unroll=False)

            pltpu.sync_copy(m_v, acc_sh.at[dst_v], add=True)
            pltpu.sync_copy(ohs[b], deg_sh.at[drow_v], add=True)

            @pl.when(t + 2 < NCHUNK)
            def _():
                issue(t + 2, b)
        return 0

    lax.fori_loop(0, NCHUNK // 2, chunk_pair, 0, unroll=False)

    plsc.subcore_barrier()

    @pl.when(s < 10)
    def _():
        pltpu.sync_copy(acc_sh.at[pl.ds(s * 1000, 1000)],
                        acc_out.at[c, pl.ds(s * 1000, 1000)])

    @pl.when(s == 11)
    def _():
        pltpu.sync_copy(deg_sh, degw_out.at[c])


_sc_edge_pass = pl.kernel(
    _sc_body,
    out_type=[
        jax.ShapeDtypeStruct((NC, N, D), jnp.float32),
        jax.ShapeDtypeStruct((NC, ND, D), jnp.float32),
    ],
    mesh=plsc.VectorSubcoreMesh(
        core_axis_name="c", subcore_axis_name="s", num_cores=NC,
        num_subcores=NS,
    ),
    scratch_types=[
        pltpu.VMEM((C,), jnp.int32),
        pltpu.VMEM((C,), jnp.int32),
        pltpu.VMEM((C,), jnp.int32),
        pltpu.VMEM((C,), jnp.int32),
        pltpu.VMEM((C,), jnp.int32),
        pltpu.VMEM((C,), jnp.int32),
        pltpu.VMEM((C,), jnp.int32),
        pltpu.VMEM((C * R,), jnp.float32),
        pltpu.VMEM((C * R,), jnp.float32),
        pltpu.VMEM((C, R * D), jnp.float32),
        pltpu.VMEM((C, R * D), jnp.float32),
        pltpu.VMEM((C, D), jnp.float32),
        pltpu.VMEM((C, D), jnp.float32),
        pltpu.VMEM((C, D), jnp.float32),
        pltpu.VMEM_SHARED((NA, D), jnp.float32),
        pltpu.VMEM_SHARED((ND, D), jnp.float32),
        pltpu.SemaphoreType.DMA,
        pltpu.SemaphoreType.DMA,
        pltpu.SemaphoreType.DMA,
        pltpu.SemaphoreType.DMA,
    ],
)


def _combine_body(a0_ref, a1_ref, d_ref, s_ref, o_ref):
    deg = jnp.maximum(d_ref[...], 1.0)
    o_ref[...] = (a0_ref[...] + a1_ref[...]) / deg + s_ref[...]


def _combine(a0, a1, d, skip):
    return pl.pallas_call(
        _combine_body,
        grid=(N // _BM,),
        in_specs=[
            pl.BlockSpec((_BM, D), lambda i: (i, 0)),
            pl.BlockSpec((_BM, D), lambda i: (i, 0)),
            pl.BlockSpec((_BM, 1), lambda i: (i, 0)),
            pl.BlockSpec((_BM, D), lambda i: (i, 0)),
        ],
        out_specs=pl.BlockSpec((_BM, D), lambda i: (i, 0)),
        out_shape=jax.ShapeDtypeStruct((N, D), jnp.float32),
    )(a0, a1, d, skip)


@jax.jit
def kernel(node_feats, edge_weights, rel_fcs, skip_w, skip_b, edge_index):
    wcat = rel_fcs.transpose(1, 0, 2).reshape(D, R * D)
    y, skip = _premultiply(node_feats, wcat, skip_w, skip_b.reshape(1, D))

    npad = EPAD - E
    src = jnp.concatenate([edge_index[0], jnp.zeros((npad,), jnp.int32)])
    dst = jnp.concatenate([edge_index[1], jnp.full((npad,), N, jnp.int32)])
    wt = jnp.concatenate(
        [edge_weights.T, jnp.zeros((npad, R), jnp.float32)]
    ).reshape(EPAD * R)

    z2d = jnp.zeros((1000, D), jnp.float32)
    oh = jnp.eye(D, dtype=jnp.float32)
    acc, degw = _sc_edge_pass(y, src, dst, wt, z2d, oh)

    deg = (degw[0] + degw[1]).reshape(ND * D)[:N].reshape(N, 1)
    return _combine(acc[0], acc[1], deg, skip)


# speedup vs baseline: 1.2769x; 1.0017x over previous
import jax
import jax.numpy as jnp
from jax import lax
from jax.experimental import pallas as pl
from jax.experimental.pallas import tpu as pltpu
from jax.experimental.pallas import tpu_sc as plsc

N = 10000
E = 320000
D = 128
R = 8

NC = 2
NS = 16
NW = NC * NS
L = 16

C = 16
EPT = 10016
EPAD = EPT * NW
NCHUNK = EPT // C
NA = N + 16
ND = 80


_BM = 400


def _mm_body(x_ref, wcat_ref, wskip_ref, b_ref, y_ref, s_ref):
    x = x_ref[...]
    y_ref[...] = jnp.dot(x, wcat_ref[...], preferred_element_type=jnp.float32)
    s_ref[...] = (
        jnp.dot(x, wskip_ref[...], preferred_element_type=jnp.float32)
        + b_ref[...]
    )


def _premultiply(x, wcat, wskip, b2d):
    return pl.pallas_call(
        _mm_body,
        grid=(N // _BM,),
        in_specs=[
            pl.BlockSpec((_BM, D), lambda i: (i, 0)),
            pl.BlockSpec((D, R * D), lambda i: (0, 0)),
            pl.BlockSpec((D, D), lambda i: (0, 0)),
            pl.BlockSpec((1, D), lambda i: (0, 0)),
        ],
        out_specs=[
            pl.BlockSpec((_BM, R * D), lambda i: (i, 0)),
            pl.BlockSpec((_BM, D), lambda i: (i, 0)),
        ],
        out_shape=[
            jax.ShapeDtypeStruct((N, R * D), jnp.float32),
            jax.ShapeDtypeStruct((N, D), jnp.float32),
        ],
    )(x, wcat, wskip, b2d)


def _sc_body(y_hbm, src_hbm, dst_hbm, wt_hbm, z2d_hbm, oh_hbm,
             acc_out, degw_out,
             src0, src1, dst0, dst1, drow_v, dmod0, dmod1, w0, w1,
             rows0, rows1, m_v, oh0, oh1, acc_sh, deg_sh,
             sem0, sem1, sem2, sem3):
    c = lax.axis_index("c")
    s = lax.axis_index("s")
    wid = c * NS + s

    srcs, dsts, ws_, rows_, sems = (
        (src0, src1), (dst0, dst1), (w0, w1), (rows0, rows1), (sem0, sem1)
    )
    ohs, osems, dmods = (oh0, oh1), (sem2, sem3), (dmod0, dmod1)

    def issue(t, b):
        base = wid * EPT + t * C
        pltpu.sync_copy(src_hbm.at[pl.ds(base, C)], srcs[b])
        pltpu.sync_copy(dst_hbm.at[pl.ds(base, C)], dsts[b])
        pltpu.sync_copy(wt_hbm.at[pl.ds(base * R, C * R)], ws_[b])
        pltpu.async_copy(y_hbm.at[srcs[b]], rows_[b], sems[b])
        dmods[b][...] = jnp.bitwise_and(dsts[b][...], 127)
        pltpu.async_copy(oh_hbm.at[dmods[b]], ohs[b], osems[b])

    issue(0, 0)
    issue(1, 1)

    @pl.when(s < 10)
    def _():
        pltpu.sync_copy(z2d_hbm, acc_sh.at[pl.ds(s * 1000, 1000)])

    @pl.when(s == 10)
    def _():
        pltpu.sync_copy(z2d_hbm.at[pl.ds(0, 16)], acc_sh.at[pl.ds(N, 16)])

    @pl.when(s == 11)
    def _():
        pltpu.sync_copy(z2d_hbm.at[pl.ds(0, ND)], deg_sh)

    plsc.subcore_barrier()

    def chunk_pair(tt, _):
        for b in range(2):
            t = tt * 2 + b
            src_v, dst_v, w_v, rows_v, sem = (
                srcs[b], dsts[b], ws_[b], rows_[b], sems[b]
            )
            pltpu.make_async_copy(y_hbm.at[src_v], rows_v, sem).wait()
            pltpu.make_async_copy(oh_hbm.at[dmods[b]], ohs[b], osems[b]).wait()

            drow_v[...] = lax.shift_right_logical(dst_v[...], 7)

            def pair_body(p, _):
                w16 = w_v[pl.ds(p * 2 * R, L)]
                for e in range(2):
                    i = p * 2 + e
                    m = [None] * (D // L)
                    for r in range(R):
                        wb = jnp.broadcast_to(w16[e * R + r], (L,))
                        for j in range(D // L):
                            v = rows_v[i, pl.ds(r * D + j * L, L)] * wb
                            m[j] = v if r == 0 else m[j] + v
                    for j in range(D // L):
                        m_v[i, pl.ds(j * L, L)] = m[j]
                return 0

            lax.fori_loop(0, C // 2, pair_body, 0, unroll=2)

            pltpu.sync_copy(m_v, acc_sh.at[dst_v], add=True)
            pltpu.sync_copy(ohs[b], deg_sh.at[drow_v], add=True)

            @pl.when(t + 2 < NCHUNK)
            def _():
                issue(t + 2, b)
        return 0

    lax.fori_loop(0, NCHUNK // 2, chunk_pair, 0, unroll=False)

    plsc.subcore_barrier()

    @pl.when(s < 10)
    def _():
        pltpu.sync_copy(acc_sh.at[pl.ds(s * 1000, 1000)],
                        acc_out.at[c, pl.ds(s * 1000, 1000)])

    @pl.when(s == 11)
    def _():
        pltpu.sync_copy(deg_sh, degw_out.at[c])


_sc_edge_pass = pl.kernel(
    _sc_body,
    out_type=[
        jax.ShapeDtypeStruct((NC, N, D), jnp.float32),
        jax.ShapeDtypeStruct((NC, ND, D), jnp.float32),
    ],
    mesh=plsc.VectorSubcoreMesh(
        core_axis_name="c", subcore_axis_name="s", num_cores=NC,
        num_subcores=NS,
    ),
    scratch_types=[
        pltpu.VMEM((C,), jnp.int32),
        pltpu.VMEM((C,), jnp.int32),
        pltpu.VMEM((C,), jnp.int32),
        pltpu.VMEM((C,), jnp.int32),
        pltpu.VMEM((C,), jnp.int32),
        pltpu.VMEM((C,), jnp.int32),
        pltpu.VMEM((C,), jnp.int32),
        pltpu.VMEM((C * R,), jnp.float32),
        pltpu.VMEM((C * R,), jnp.float32),
        pltpu.VMEM((C, R * D), jnp.float32),
        pltpu.VMEM((C, R * D), jnp.float32),
        pltpu.VMEM((C, D), jnp.float32),
        pltpu.VMEM((C, D), jnp.float32),
        pltpu.VMEM((C, D), jnp.float32),
        pltpu.VMEM_SHARED((NA, D), jnp.float32),
        pltpu.VMEM_SHARED((ND, D), jnp.float32),
        pltpu.SemaphoreType.DMA,
        pltpu.SemaphoreType.DMA,
        pltpu.SemaphoreType.DMA,
        pltpu.SemaphoreType.DMA,
    ],
)


def _combine_body(a0_ref, a1_ref, d_ref, s_ref, o_ref):
    deg = jnp.maximum(d_ref[...], 1.0)
    o_ref[...] = (a0_ref[...] + a1_ref[...]) / deg + s_ref[...]


def _combine(a0, a1, d, skip):
    return pl.pallas_call(
        _combine_body,
        grid=(N // _BM,),
        in_specs=[
            pl.BlockSpec((_BM, D), lambda i: (i, 0)),
            pl.BlockSpec((_BM, D), lambda i: (i, 0)),
            pl.BlockSpec((_BM, 1), lambda i: (i, 0)),
            pl.BlockSpec((_BM, D), lambda i: (i, 0)),
        ],
        out_specs=pl.BlockSpec((_BM, D), lambda i: (i, 0)),
        out_shape=jax.ShapeDtypeStruct((N, D), jnp.float32),
    )(a0, a1, d, skip)


@jax.jit
def kernel(node_feats, edge_weights, rel_fcs, skip_w, skip_b, edge_index):
    wcat = rel_fcs.transpose(1, 0, 2).reshape(D, R * D)
    y, skip = _premultiply(node_feats, wcat, skip_w, skip_b.reshape(1, D))

    npad = EPAD - E
    src = jnp.concatenate([edge_index[0], jnp.zeros((npad,), jnp.int32)])
    dst = jnp.concatenate([edge_index[1], jnp.full((npad,), N, jnp.int32)])
    wt = jnp.concatenate(
        [edge_weights.T, jnp.zeros((npad, R), jnp.float32)]
    ).reshape(EPAD * R)

    z2d = jnp.zeros((1000, D), jnp.float32)
    oh = jnp.eye(D, dtype=jnp.float32)
    acc, degw = _sc_edge_pass(y, src, dst, wt, z2d, oh)

    deg = (degw[0] + degw[1]).reshape(ND * D)[:N].reshape(N, 1)
    return _combine(acc[0], acc[1], deg, skip)


# speedup vs baseline: 1.2832x; 1.0049x over previous
import jax
import jax.numpy as jnp
from jax import lax
from jax.experimental import pallas as pl
from jax.experimental.pallas import tpu as pltpu
from jax.experimental.pallas import tpu_sc as plsc

N = 10000
E = 320000
D = 128
R = 8

NC = 2
NS = 16
NW = NC * NS
L = 16

C = 16
EPT = 10016
EPAD = EPT * NW
NCHUNK = EPT // C
NA = N + 16
ND = 80


_BM = 400


def _mm_body(x_ref, wcat_ref, wskip_ref, b_ref, y_ref, s_ref):
    x = x_ref[...]
    y = jnp.dot(x, wcat_ref[...], preferred_element_type=jnp.float32)
    u = lax.bitcast_convert_type(
        y.astype(jnp.bfloat16), jnp.uint16
    ).astype(jnp.uint32)
    lo = [u[:, r * D: r * D + D // 2] for r in range(R)]
    hi = [u[:, r * D + D // 2: (r + 1) * D] for r in range(R)]
    packed = jnp.concatenate(
        [lo[r] | (hi[r] << 16) for r in range(R)], axis=1
    )
    y_ref[...] = lax.bitcast_convert_type(packed, jnp.int32)
    s_ref[...] = (
        jnp.dot(x, wskip_ref[...], preferred_element_type=jnp.float32)
        + b_ref[...]
    )


def _premultiply(x, wcat, wskip, b2d):
    return pl.pallas_call(
        _mm_body,
        grid=(N // _BM,),
        in_specs=[
            pl.BlockSpec((_BM, D), lambda i: (i, 0)),
            pl.BlockSpec((D, R * D), lambda i: (0, 0)),
            pl.BlockSpec((D, D), lambda i: (0, 0)),
            pl.BlockSpec((1, D), lambda i: (0, 0)),
        ],
        out_specs=[
            pl.BlockSpec((_BM, R * D // 2), lambda i: (i, 0)),
            pl.BlockSpec((_BM, D), lambda i: (i, 0)),
        ],
        out_shape=[
            jax.ShapeDtypeStruct((N, R * D // 2), jnp.int32),
            jax.ShapeDtypeStruct((N, D), jnp.float32),
        ],
    )(x, wcat, wskip, b2d)


def _sc_body(y_hbm, src_hbm, dst_hbm, wt_hbm, z2d_hbm, oh_hbm,
             acc_out, degw_out,
             src0, src1, dst0, dst1, drow_v, dmod0, dmod1, w0, w1,
             rows0, rows1, m_v, oh0, oh1, acc_sh, deg_sh,
             sem0, sem1, sem2, sem3):
    c = lax.axis_index("c")
    s = lax.axis_index("s")
    wid = c * NS + s

    srcs, dsts, ws_, rows_, sems = (
        (src0, src1), (dst0, dst1), (w0, w1), (rows0, rows1), (sem0, sem1)
    )
    ohs, osems, dmods = (oh0, oh1), (sem2, sem3), (dmod0, dmod1)

    def issue(t, b):
        base = wid * EPT + t * C
        pltpu.sync_copy(src_hbm.at[pl.ds(base, C)], srcs[b])
        pltpu.sync_copy(dst_hbm.at[pl.ds(base, C)], dsts[b])
        pltpu.sync_copy(wt_hbm.at[pl.ds(base * R, C * R)], ws_[b])
        pltpu.async_copy(y_hbm.at[srcs[b]], rows_[b], sems[b])
        dmods[b][...] = jnp.bitwise_and(dsts[b][...], 127)
        pltpu.async_copy(oh_hbm.at[dmods[b]], ohs[b], osems[b])

    issue(0, 0)
    issue(1, 1)

    @pl.when(s < 10)
    def _():
        pltpu.sync_copy(z2d_hbm, acc_sh.at[pl.ds(s * 1000, 1000)])

    @pl.when(s == 10)
    def _():
        pltpu.sync_copy(z2d_hbm.at[pl.ds(0, 16)], acc_sh.at[pl.ds(N, 16)])

    @pl.when(s == 11)
    def _():
        pltpu.sync_copy(z2d_hbm.at[pl.ds(0, ND)], deg_sh)

    plsc.subcore_barrier()

    def chunk_pair(tt, _):
        for b in range(2):
            t = tt * 2 + b
            src_v, dst_v, w_v, rows_v, sem = (
                srcs[b], dsts[b], ws_[b], rows_[b], sems[b]
            )
            pltpu.make_async_copy(y_hbm.at[src_v], rows_v, sem).wait()
            pltpu.make_async_copy(oh_hbm.at[dmods[b]], ohs[b], osems[b]).wait()

            drow_v[...] = lax.shift_right_logical(dst_v[...], 7)

            def pair_body(p, _):
                w16 = w_v[pl.ds(p * 2 * R, L)]
                for e in range(2):
                    i = p * 2 + e
                    m = [None] * (D // L)
                    for r in range(R):
                        wb = jnp.broadcast_to(w16[e * R + r], (L,))
                        for j in range(D // (2 * L)):
                            v = rows_v[i, pl.ds(r * (D // 2) + j * L, L)]
                            flo = lax.bitcast_convert_type(
                                v << 16, jnp.float32) * wb
                            fhi = lax.bitcast_convert_type(
                                v & jnp.int32(-65536), jnp.float32) * wb
                            jh = j + D // (2 * L)
                            m[j] = flo if r == 0 else m[j] + flo
                            m[jh] = fhi if r == 0 else m[jh] + fhi
                    for j in range(D // L):
                        m_v[i, pl.ds(j * L, L)] = m[j]
                return 0

            lax.fori_loop(0, C // 2, pair_body, 0, unroll=False)

            pltpu.sync_copy(m_v, acc_sh.at[dst_v], add=True)
            pltpu.sync_copy(ohs[b], deg_sh.at[drow_v], add=True)

            @pl.when(t + 2 < NCHUNK)
            def _():
                issue(t + 2, b)
        return 0

    lax.fori_loop(0, NCHUNK // 2, chunk_pair, 0, unroll=False)

    plsc.subcore_barrier()

    @pl.when(s < 10)
    def _():
        pltpu.sync_copy(acc_sh.at[pl.ds(s * 1000, 1000)],
                        acc_out.at[c, pl.ds(s * 1000, 1000)])

    @pl.when(s == 11)
    def _():
        pltpu.sync_copy(deg_sh, degw_out.at[c])


_sc_edge_pass = pl.kernel(
    _sc_body,
    out_type=[
        jax.ShapeDtypeStruct((NC, N, D), jnp.float32),
        jax.ShapeDtypeStruct((NC, ND, D), jnp.float32),
    ],
    mesh=plsc.VectorSubcoreMesh(
        core_axis_name="c", subcore_axis_name="s", num_cores=NC,
        num_subcores=NS,
    ),
    scratch_types=[
        pltpu.VMEM((C,), jnp.int32),
        pltpu.VMEM((C,), jnp.int32),
        pltpu.VMEM((C,), jnp.int32),
        pltpu.VMEM((C,), jnp.int32),
        pltpu.VMEM((C,), jnp.int32),
        pltpu.VMEM((C,), jnp.int32),
        pltpu.VMEM((C,), jnp.int32),
        pltpu.VMEM((C * R,), jnp.float32),
        pltpu.VMEM((C * R,), jnp.float32),
        pltpu.VMEM((C, R * D // 2), jnp.int32),
        pltpu.VMEM((C, R * D // 2), jnp.int32),
        pltpu.VMEM((C, D), jnp.float32),
        pltpu.VMEM((C, D), jnp.float32),
        pltpu.VMEM((C, D), jnp.float32),
        pltpu.VMEM_SHARED((NA, D), jnp.float32),
        pltpu.VMEM_SHARED((ND, D), jnp.float32),
        pltpu.SemaphoreType.DMA,
        pltpu.SemaphoreType.DMA,
        pltpu.SemaphoreType.DMA,
        pltpu.SemaphoreType.DMA,
    ],
)


def _combine_body(a0_ref, a1_ref, d_ref, s_ref, o_ref):
    deg = jnp.maximum(d_ref[...], 1.0)
    o_ref[...] = (a0_ref[...] + a1_ref[...]) / deg + s_ref[...]


def _combine(a0, a1, d, skip):
    return pl.pallas_call(
        _combine_body,
        grid=(N // _BM,),
        in_specs=[
            pl.BlockSpec((_BM, D), lambda i: (i, 0)),
            pl.BlockSpec((_BM, D), lambda i: (i, 0)),
            pl.BlockSpec((_BM, 1), lambda i: (i, 0)),
            pl.BlockSpec((_BM, D), lambda i: (i, 0)),
        ],
        out_specs=pl.BlockSpec((_BM, D), lambda i: (i, 0)),
        out_shape=jax.ShapeDtypeStruct((N, D), jnp.float32),
    )(a0, a1, d, skip)


@jax.jit
def kernel(node_feats, edge_weights, rel_fcs, skip_w, skip_b, edge_index):
    wcat = rel_fcs.transpose(1, 0, 2).reshape(D, R * D)
    y, skip = _premultiply(node_feats, wcat, skip_w, skip_b.reshape(1, D))

    npad = EPAD - E
    src = jnp.concatenate([edge_index[0], jnp.zeros((npad,), jnp.int32)])
    dst = jnp.concatenate([edge_index[1], jnp.full((npad,), N, jnp.int32)])
    wt = jnp.concatenate(
        [edge_weights.T, jnp.zeros((npad, R), jnp.float32)]
    ).reshape(EPAD * R)

    z2d = jnp.zeros((1000, D), jnp.float32)
    oh = jnp.eye(D, dtype=jnp.float32)
    acc, degw = _sc_edge_pass(y, src, dst, wt, z2d, oh)

    deg = (degw[0] + degw[1]).reshape(ND * D)[:N].reshape(N, 1)
    return _combine(acc[0], acc[1], deg, skip)


# speedup vs baseline: 2.2915x; 1.7858x over previous
import jax
import jax.numpy as jnp
from jax import lax
from jax.experimental import pallas as pl
from jax.experimental.pallas import tpu as pltpu
from jax.experimental.pallas import tpu_sc as plsc

N = 10000
E = 320000
D = 128
R = 8

NC = 2
NS = 16
NW = NC * NS
L = 16

C = 16
EPT = 10016
EPAD = EPT * NW
NCHUNK = EPT // C
NA = N + 16
ND = 80


_BM = 400


def _mm_body(x_ref, wcat_ref, wskip_ref, b_ref, y_ref, s_ref):
    x = x_ref[...]
    y = jnp.dot(x, wcat_ref[...], preferred_element_type=jnp.float32)
    u = lax.bitcast_convert_type(
        y.astype(jnp.bfloat16), jnp.uint16
    ).astype(jnp.uint32)
    lo = [u[:, r * D: r * D + D // 2] for r in range(R)]
    hi = [u[:, r * D + D // 2: (r + 1) * D] for r in range(R)]
    packed = jnp.concatenate(
        [lo[r] | (hi[r] << 16) for r in range(R)], axis=1
    )
    y_ref[...] = lax.bitcast_convert_type(packed, jnp.int32)
    s_ref[...] = (
        jnp.dot(x, wskip_ref[...], preferred_element_type=jnp.float32)
        + b_ref[...]
    )


def _premultiply(x, wcat, wskip, b2d):
    return pl.pallas_call(
        _mm_body,
        grid=(N // _BM,),
        in_specs=[
            pl.BlockSpec((_BM, D), lambda i: (i, 0)),
            pl.BlockSpec((D, R * D), lambda i: (0, 0)),
            pl.BlockSpec((D, D), lambda i: (0, 0)),
            pl.BlockSpec((1, D), lambda i: (0, 0)),
        ],
        out_specs=[
            pl.BlockSpec((_BM, R * D // 2), lambda i: (i, 0)),
            pl.BlockSpec((_BM, D), lambda i: (i, 0)),
        ],
        out_shape=[
            jax.ShapeDtypeStruct((N, R * D // 2), jnp.int32),
            jax.ShapeDtypeStruct((N, D), jnp.float32),
        ],
    )(x, wcat, wskip, b2d)


def _sc_body(y_hbm, src_hbm, dst_hbm, wt_hbm, z2d_hbm, oh_hbm,
             acc_out, degw_out,
             src0, src1, src2, src3, dst0, dst1, dst2, dst3,
             w0, w1, w2, w3, drow_v, dmod0, dmod1,
             rows0, rows1, m_v, oh0, oh1, acc_sh, deg_sh,
             msem0, msem1, msem2, msem3, rsem0, rsem1, osem0, osem1):
    c = lax.axis_index("c")
    s = lax.axis_index("s")
    wid = c * NS + s

    srcs, dsts, ws_ = (src0, src1, src2, src3), (dst0, dst1, dst2, dst3), \
        (w0, w1, w2, w3)
    msems = (msem0, msem1, msem2, msem3)
    rows_, rsems = (rows0, rows1), (rsem0, rsem1)
    ohs, osems, dmods = (oh0, oh1), (osem0, osem1), (dmod0, dmod1)

    def meta_issue(t, q):
        base = wid * EPT + t * C
        pltpu.async_copy(src_hbm.at[pl.ds(base, C)], srcs[q], msems[q])
        pltpu.async_copy(dst_hbm.at[pl.ds(base, C)], dsts[q], msems[q])
        pltpu.async_copy(wt_hbm.at[pl.ds(base * R, C * R)], ws_[q], msems[q])

    def rows_issue(t, q, par):
        base = wid * EPT + t * C
        pltpu.make_async_copy(
            src_hbm.at[pl.ds(base, C)], srcs[q], msems[q]).wait()
        pltpu.make_async_copy(
            dst_hbm.at[pl.ds(base, C)], dsts[q], msems[q]).wait()
        pltpu.make_async_copy(
            wt_hbm.at[pl.ds(base * R, C * R)], ws_[q], msems[q]).wait()
        pltpu.async_copy(y_hbm.at[srcs[q]], rows_[par], rsems[par])
        dmods[par][...] = jnp.bitwise_and(dsts[q][...], 127)
        pltpu.async_copy(oh_hbm.at[dmods[par]], ohs[par], osems[par])

    for t0 in range(4):
        meta_issue(t0, t0)
    rows_issue(0, 0, 0)
    rows_issue(1, 1, 1)

    @pl.when(s < 10)
    def _():
        pltpu.sync_copy(z2d_hbm, acc_sh.at[pl.ds(s * 1000, 1000)])

    @pl.when(s == 10)
    def _():
        pltpu.sync_copy(z2d_hbm.at[pl.ds(0, 16)], acc_sh.at[pl.ds(N, 16)])

    @pl.when(s == 11)
    def _():
        pltpu.sync_copy(z2d_hbm.at[pl.ds(0, ND)], deg_sh)

    plsc.subcore_barrier()

    def process(t, q, par, tail):
        dst_v, w_v, rows_v = dsts[q], ws_[q], rows_[par]
        pltpu.make_async_copy(y_hbm.at[srcs[q]], rows_v, rsems[par]).wait()
        pltpu.make_async_copy(
            oh_hbm.at[dmods[par]], ohs[par], osems[par]).wait()

        drow_v[...] = lax.shift_right_logical(dst_v[...], 7)

        def pair_body(p, _):
            w16 = w_v[pl.ds(p * 2 * R, L)]
            for e in range(2):
                i = p * 2 + e
                m = [None] * (D // L)
                for r in range(R):
                    wb = jnp.broadcast_to(w16[e * R + r], (L,))
                    for j in range(D // (2 * L)):
                        v = rows_v[i, pl.ds(r * (D // 2) + j * L, L)]
                        flo = lax.bitcast_convert_type(
                            v << 16, jnp.float32) * wb
                        fhi = lax.bitcast_convert_type(
                            v & jnp.int32(-65536), jnp.float32) * wb
                        jh = j + D // (2 * L)
                        m[j] = flo if r == 0 else m[j] + flo
                        m[jh] = fhi if r == 0 else m[jh] + fhi
                for j in range(D // L):
                    m_v[i, pl.ds(j * L, L)] = m[j]
            return 0

        lax.fori_loop(0, C // 2, pair_body, 0, unroll=False)

        pltpu.sync_copy(m_v, acc_sh.at[dst_v], add=True)
        pltpu.sync_copy(ohs[par], deg_sh.at[drow_v], add=True)

        if not tail:
            @pl.when(t + 4 < NCHUNK)
            def _():
                meta_issue(t + 4, q)

            rows_issue(t + 2, (q + 2) % 4, par)

    def quad(tt, _):
        for b in range(4):
            process(tt * 4 + b, b, b % 2, False)
        return 0

    lax.fori_loop(0, NCHUNK // 4, quad, 0, unroll=False)
    process(NCHUNK - 2, (NCHUNK - 2) % 4, 0, True)
    process(NCHUNK - 1, (NCHUNK - 1) % 4, 1, True)

    plsc.subcore_barrier()

    @pl.when(s < 10)
    def _():
        pltpu.sync_copy(acc_sh.at[pl.ds(s * 1000, 1000)],
                        acc_out.at[c, pl.ds(s * 1000, 1000)])

    @pl.when(s == 11)
    def _():
        pltpu.sync_copy(deg_sh, degw_out.at[c])


_sc_edge_pass = pl.kernel(
    _sc_body,
    out_type=[
        jax.ShapeDtypeStruct((NC, N, D), jnp.float32),
        jax.ShapeDtypeStruct((NC, ND, D), jnp.float32),
    ],
    mesh=plsc.VectorSubcoreMesh(
        core_axis_name="c", subcore_axis_name="s", num_cores=NC,
        num_subcores=NS,
    ),
    scratch_types=(
        [pltpu.VMEM((C,), jnp.int32) for _ in range(4)]
        + [pltpu.VMEM((C,), jnp.int32) for _ in range(4)]
        + [pltpu.VMEM((C * R,), jnp.float32) for _ in range(4)]
        + [
            pltpu.VMEM((C,), jnp.int32),
            pltpu.VMEM((C,), jnp.int32),
            pltpu.VMEM((C,), jnp.int32),
            pltpu.VMEM((C, R * D // 2), jnp.int32),
            pltpu.VMEM((C, R * D // 2), jnp.int32),
            pltpu.VMEM((C, D), jnp.float32),
            pltpu.VMEM((C, D), jnp.float32),
            pltpu.VMEM((C, D), jnp.float32),
            pltpu.VMEM_SHARED((NA, D), jnp.float32),
            pltpu.VMEM_SHARED((ND, D), jnp.float32),
        ]
        + [pltpu.SemaphoreType.DMA for _ in range(8)]
    ),
)


def _combine_body(a0_ref, a1_ref, d_ref, s_ref, o_ref):
    deg = jnp.maximum(d_ref[...], 1.0)
    o_ref[...] = (a0_ref[...] + a1_ref[...]) / deg + s_ref[...]


def _combine(a0, a1, d, skip):
    return pl.pallas_call(
        _combine_body,
        grid=(N // _BM,),
        in_specs=[
            pl.BlockSpec((_BM, D), lambda i: (i, 0)),
            pl.BlockSpec((_BM, D), lambda i: (i, 0)),
            pl.BlockSpec((_BM, 1), lambda i: (i, 0)),
            pl.BlockSpec((_BM, D), lambda i: (i, 0)),
        ],
        out_specs=pl.BlockSpec((_BM, D), lambda i: (i, 0)),
        out_shape=jax.ShapeDtypeStruct((N, D), jnp.float32),
    )(a0, a1, d, skip)


@jax.jit
def kernel(node_feats, edge_weights, rel_fcs, skip_w, skip_b, edge_index):
    wcat = rel_fcs.transpose(1, 0, 2).reshape(D, R * D)
    y, skip = _premultiply(node_feats, wcat, skip_w, skip_b.reshape(1, D))

    npad = EPAD - E
    src = jnp.concatenate([edge_index[0], jnp.zeros((npad,), jnp.int32)])
    dst = jnp.concatenate([edge_index[1], jnp.full((npad,), N, jnp.int32)])
    wt = jnp.concatenate(
        [edge_weights.T, jnp.zeros((npad, R), jnp.float32)]
    ).reshape(EPAD * R)

    z2d = jnp.zeros((1000, D), jnp.float32)
    oh = jnp.eye(D, dtype=jnp.float32)
    acc, degw = _sc_edge_pass(y, src, dst, wt, z2d, oh)

    deg = (degw[0] + degw[1]).reshape(ND * D)[:N].reshape(N, 1)
    return _combine(acc[0], acc[1], deg, skip)
